# Initial kernel scaffold; baseline (speedup 1.0000x reference)
import jax
import jax.numpy as jnp
from jax.experimental import pallas as pl


def kernel(x, edge_index, edge_values, W1, b1, W2, b2):
    raise NotImplementedError("write your pallas kernel here")


# speedup vs baseline: 11.0079x; 11.0079x over previous
import functools

import jax
import jax.numpy as jnp
from jax import lax
from jax.experimental import pallas as pl
from jax.experimental.pallas import tpu as pltpu
from jax.experimental.pallas import tpu_sc as plsc

NC = 2
NS = 16
NW = NC * NS
LANE = 16


def _mm_body(x_ref, w_ref, o_ref):
    o_ref[...] = jnp.dot(x_ref[...], w_ref[...],
                         preferred_element_type=jnp.float32
                         ).astype(jnp.bfloat16)


def _combine_mm_body(p_ref, b_ref, w_ref, o_ref):
    xb = jnp.maximum(p_ref[0] + p_ref[1] + b_ref[...], 0.0)
    o_ref[...] = jnp.dot(xb, w_ref[...], preferred_element_type=jnp.float32
                         ).astype(jnp.bfloat16)


def _combine_body(p_ref, b_ref, pm_ref, o_ref):
    xb = jnp.maximum(p_ref[0] + p_ref[1] + b_ref[...], 0.0)
    o_ref[...] = jnp.dot(xb, pm_ref[...], preferred_element_type=jnp.float32)


def _tc_mm(x, w, br):
    n, d = x.shape
    return pl.pallas_call(
        _mm_body,
        grid=(n // br,),
        in_specs=[pl.BlockSpec((br, d), lambda i: (i, 0)),
                  pl.BlockSpec((d, d), lambda i: (0, 0))],
        out_specs=pl.BlockSpec((br, d), lambda i: (i, 0)),
        out_shape=jax.ShapeDtypeStruct((n, d), jnp.bfloat16),
    )(x, w)


def _tc_combine_mm(parts, b, w, br, n):
    d = parts.shape[-1]
    return pl.pallas_call(
        _combine_mm_body,
        grid=(n // br,),
        in_specs=[pl.BlockSpec((2, br, d), lambda i: (0, i, 0)),
                  pl.BlockSpec((1, d), lambda i: (0, 0)),
                  pl.BlockSpec((d, d), lambda i: (0, 0))],
        out_specs=pl.BlockSpec((br, d), lambda i: (i, 0)),
        out_shape=jax.ShapeDtypeStruct((n, d), jnp.bfloat16),
    )(parts, b.reshape(1, d), w)


def _tc_combine(parts, b, pm, br, n):
    d = parts.shape[-1]
    return pl.pallas_call(
        _combine_body,
        grid=(n // br,),
        in_specs=[pl.BlockSpec((2, br, d), lambda i: (0, i, 0)),
                  pl.BlockSpec((1, d), lambda i: (0, 0)),
                  pl.BlockSpec((d, d), lambda i: (0, 0))],
        out_specs=pl.BlockSpec((br, d), lambda i: (i, 0)),
        out_shape=jax.ShapeDtypeStruct((n, d), jnp.float32),
    )(parts, b.reshape(1, d), pm)


def _bcast_lane(vec, lane):
    idx = jnp.full((LANE, 1), lane, jnp.int32)
    dnums = lax.GatherDimensionNumbers(
        offset_dims=(), collapsed_slice_dims=(0,), start_index_map=(0,))
    return lax.gather(vec, idx, dnums, (1,),
                      mode=lax.GatherScatterMode.PROMISE_IN_BOUNDS)


@functools.cache
def _make_sc_agg(n, d, e, k):
    ep = e // NW
    nchunk = ep // k
    np_ = -(-n // (NS * 8)) * (NS * 8)
    rp = np_ // NS
    zb = 8
    mesh = plsc.VectorSubcoreMesh(core_axis_name="c", subcore_axis_name="s",
                                  num_cores=NC, num_subcores=NS)

    def body(h_hbm, row_hbm, col_hbm, val_hbm, out_hbm,
             acc, rowc, colc, valb, rows_v, msg_v, zeros_v,
             gsem, ssem, vsem, zsem):
        c = lax.axis_index("c")
        s = lax.axis_index("s")
        wid = c * NS + s
        for i in range(zb):
            for j in range(d // LANE):
                zeros_v[i, pl.ds(j * LANE, LANE)] = jnp.zeros((LANE,),
                                                              jnp.float32)
        r0 = s * rp
        for t in range(rp // zb):
            pltpu.async_copy(zeros_v, acc.at[pl.ds(r0 + t * zb, zb)],
                             zsem)
        for t in range(rp // zb):
            pltpu.make_async_copy(zeros_v, acc.at[pl.ds(r0, zb)],
                                  zsem).wait()
        pltpu.async_copy(row_hbm.at[wid], rowc, zsem)
        pltpu.async_copy(col_hbm.at[wid], colc, zsem)
        pltpu.make_async_copy(row_hbm.at[wid], rowc, zsem).wait()
        pltpu.make_async_copy(col_hbm.at[wid], colc, zsem).wait()
        plsc.subcore_barrier()

        vb0 = wid * nchunk
        for b in range(2):
            pltpu.async_copy(val_hbm.at[vb0 + b], valb.at[b], vsem.at[b])
            pltpu.async_copy(h_hbm.at[colc.at[b]], rows_v.at[b], gsem.at[b])

        def scale_and_swap(t, b):
            i = 2 * t + b
            va = lax.rem(i, 4)
            vp = lax.rem(i + 2, 4)

            @pl.when(t > 0)
            def _():
                pltpu.make_async_copy(msg_v.at[b], acc.at[rowc.at[i]],
                                      ssem.at[b]).wait()
            pltpu.make_async_copy(h_hbm.at[colc.at[i]], rows_v.at[b],
                                  gsem.at[b]).wait()
            pltpu.make_async_copy(val_hbm.at[vb0 + i], valb.at[va],
                                  vsem.at[b]).wait()

            @pl.when(t < nchunk // 2 - 1)
            def _():
                pltpu.async_copy(val_hbm.at[vb0 + i + 2], valb.at[vp],
                                 vsem.at[b])
            for g in range((k + LANE - 1) // LANE):
                base = min(g * LANE, k - LANE)
                vv = valb[va, pl.ds(base, LANE)]
                for kk in range(g * LANE, min(k, (g + 1) * LANE)):
                    vb = _bcast_lane(vv, kk - base)
                    for j in range(d // (2 * LANE)):
                        rv = rows_v[b, kk, pl.ds(j * 2 * LANE, 2 * LANE)]
                        lo, hi = plsc.unpack(
                            rv, format=plsc.PackFormat.INTERLEAVED)
                        msg_v[b, kk, pl.ds(j * 2 * LANE, LANE)] = lo * vb
                        msg_v[b, kk, pl.ds(j * 2 * LANE + LANE, LANE)] = \
                            hi * vb

            @pl.when(t < nchunk // 2 - 1)
            def _():
                pltpu.async_copy(h_hbm.at[colc.at[i + 2]], rows_v.at[b],
                                 gsem.at[b])
            pltpu.async_copy(msg_v.at[b], acc.at[rowc.at[i]], ssem.at[b],
                             add=True)

        def step(t, carry):
            scale_and_swap(t, 0)
            scale_and_swap(t, 1)
            return carry

        lax.fori_loop(0, nchunk // 2, step, 0)
        for b in range(2):
            pltpu.make_async_copy(msg_v.at[b], acc.at[rowc.at[b]],
                                  ssem.at[b]).wait()
        plsc.subcore_barrier()
        pltpu.sync_copy(acc.at[pl.ds(r0, rp)], out_hbm.at[c, pl.ds(r0, rp)])

    return pl.kernel(
        body,
        out_type=jax.ShapeDtypeStruct((NC, np_, d), jnp.float32),
        mesh=mesh,
        scratch_types=[
            pltpu.VMEM_SHARED((np_, d), jnp.float32),
            pltpu.VMEM((nchunk, k), jnp.int32),
            pltpu.VMEM((nchunk, k), jnp.int32),
            pltpu.VMEM((4, k), jnp.float32),
            pltpu.VMEM((2, k, d), jnp.bfloat16),
            pltpu.VMEM((2, k, d), jnp.float32),
            pltpu.VMEM((zb, d), jnp.float32),
            pltpu.SemaphoreType.DMA((2,)),
            pltpu.SemaphoreType.DMA((2,)),
            pltpu.SemaphoreType.DMA((2,)),
            pltpu.SemaphoreType.DMA,
        ],
        compiler_params=pltpu.CompilerParams(needs_layout_passes=False,
                                             use_tc_tiling_on_sc=False),
    )


def kernel(x, edge_index, edge_values, W1, b1, W2, b2):
    n, d = x.shape
    e = edge_values.shape[0]
    k = 50
    nchunk = e // (NW * k)
    row3 = edge_index[0].reshape(NW, nchunk, k)
    col3 = edge_index[1].reshape(NW, nchunk, k)
    val2 = edge_values.reshape(NW * nchunk, k)
    sc_agg = _make_sc_agg(n, d, e, k)
    br = n

    sig = jnp.arange(d).reshape(d // 32, 16, 2).transpose(0, 2, 1).reshape(d)
    pm = jnp.eye(d, dtype=jnp.float32)[sig]

    h1 = _tc_mm(x, W1, br)
    parts1 = sc_agg(h1, row3, col3, val2)
    h2 = _tc_combine_mm(parts1, b1[sig], W2[sig, :], br, n)
    parts2 = sc_agg(h2, row3, col3, val2)
    return _tc_combine(parts2, b2[sig], pm, br, n)
